# Initial kernel scaffold; baseline (speedup 1.0000x reference)
import jax
import jax.numpy as jnp
from jax.experimental import pallas as pl


def kernel(img, input_ids, bbox, emb_table, proj_w, stride):
    raise NotImplementedError("write your pallas kernel here")


# speedup vs baseline: 66.6939x; 66.6939x over previous
import functools

import jax
import jax.numpy as jnp
from jax import lax
from jax.experimental import pallas as pl
from jax.experimental.pallas import tpu as pltpu
from jax.experimental.pallas import tpu_sc as plsc

_NC = 2
_NS = 16
_NWORKERS = _NC * _NS


def _sc_gather_rows(emb_table, idx, n_rows, hid):
    rows_per_w = n_rows // _NWORKERS

    mesh = plsc.VectorSubcoreMesh(core_axis_name="c", subcore_axis_name="s")

    @functools.partial(
        pl.kernel,
        mesh=mesh,
        out_type=jax.ShapeDtypeStruct((n_rows, hid), jnp.float32),
        scratch_types=[
            pltpu.VMEM((rows_per_w,), jnp.int32),
            pltpu.VMEM((rows_per_w, hid), jnp.float32),
            pltpu.SemaphoreType.DMA,
        ],
    )
    def gather_kernel(table_hbm, idx_hbm, out_hbm, idx_v, rows_v, sem):
        wid = lax.axis_index("s") * _NC + lax.axis_index("c")
        base = wid * rows_per_w
        pltpu.sync_copy(idx_hbm.at[pl.ds(base, rows_per_w)], idx_v)
        pltpu.async_copy(table_hbm.at[idx_v], rows_v, sem).wait()
        pltpu.sync_copy(rows_v, out_hbm.at[pl.ds(base, rows_per_w)])

    return gather_kernel(emb_table, idx)


def _tc_kernel(words_ref, zrow_ref, bbox_ref, bbox_t_ref, proj_ref, out_ref):
    nw = words_ref.shape[0]
    h = 128
    w = 128
    ngrp = nw // 16
    kdim = nw + 8

    rows_ext = jnp.concatenate([words_ref[...], zrow_ref[...]], axis=0)
    pt = lax.dot_general(
        proj_ref[...], rows_ext,
        (((0,), (1,)), ((), ())),
        preferred_element_type=jnp.float32,
    )

    bb = jnp.rint(bbox_ref[0])
    ws_c, we_c = bb[:, 0:1], bb[:, 2:3]
    bbt = jnp.rint(bbox_t_ref[0])
    hs_r, he_r = bbt[1:2, :], bbt[3:4, :]

    n_row = lax.broadcasted_iota(jnp.int32, (1, nw), 1)
    w_row = (jnp.int32(1) << (n_row & 15)).astype(jnp.float32)
    yy = lax.broadcasted_iota(jnp.int32, (h, nw), 0).astype(jnp.float32)
    y_mat = jnp.where((yy >= hs_r) & (yy < he_r), w_row, 0.0)

    n_col = lax.broadcasted_iota(jnp.int32, (nw, 1), 0)
    jlane = lax.broadcasted_iota(jnp.int32, (nw, ngrp * w), 1)
    xf = (jlane & (w - 1)).astype(jnp.float32)
    in_grp = (n_col >> 4) == (jlane >> 7)
    x_mat = jnp.where((xf >= ws_c) & (xf < we_c) & in_grp, 1.0, 0.0)

    s = jnp.dot(y_mat, x_mat, preferred_element_type=jnp.float32)
    e = (lax.bitcast_convert_type(s, jnp.int32) >> 23) - 127
    cand = e + (lax.broadcasted_iota(jnp.int32, (h, ngrp * w), 1) >> 7) * 16
    best = cand[:, 0:w]
    for g in range(1, ngrp):
        best = jnp.maximum(best, cand[:, g * w:(g + 1) * w])
    sel = jnp.where(best >= 0, best, nw)

    sel_flat = jnp.reshape(sel, (1, h * w))
    k_iota = lax.broadcasted_iota(jnp.int32, (kdim, h * w), 0)
    oh = (k_iota == sel_flat).astype(jnp.float32)
    out_ref[0] = jnp.dot(pt, oh, preferred_element_type=jnp.float32)


def kernel(img, input_ids, bbox, emb_table, proj_w, stride):
    b, _, img_h, img_w = img.shape
    h, w = img_h // 4, img_w // 4
    nw = input_ids.shape[1]
    hid = emb_table.shape[1]
    edim = proj_w.shape[1]

    n_flat = b * nw
    n_rows = -(-(n_flat + 8) // (8 * _NWORKERS)) * (8 * _NWORKERS)
    idx = jnp.concatenate([
        input_ids.reshape(-1).astype(jnp.int32),
        jnp.zeros((n_rows - n_flat,), jnp.int32),
    ])
    rows = _sc_gather_rows(emb_table, idx, n_rows, hid)

    bbox_s = (bbox / stride).astype(jnp.float32)
    bbox_t = jnp.transpose(bbox_s, (0, 2, 1))

    out_flat = pl.pallas_call(
        _tc_kernel,
        grid=(b,),
        in_specs=[
            pl.BlockSpec((nw, hid), lambda i: (i, 0)),
            pl.BlockSpec((8, hid), lambda i: (n_flat // 8, 0)),
            pl.BlockSpec((1, nw, 4), lambda i: (i, 0, 0)),
            pl.BlockSpec((1, 4, nw), lambda i: (i, 0, 0)),
            pl.BlockSpec((hid, edim), lambda i: (0, 0)),
        ],
        out_specs=pl.BlockSpec((1, edim, h * w), lambda i: (i, 0, 0)),
        out_shape=jax.ShapeDtypeStruct((b, edim, h * w), jnp.float32),
    )(rows, rows, bbox_s, bbox_t, proj_w)

    return out_flat.reshape(b, edim, h, w)


# speedup vs baseline: 87.8494x; 1.3172x over previous
import functools

import jax
import jax.numpy as jnp
from jax import lax
from jax.experimental import pallas as pl
from jax.experimental.pallas import tpu as pltpu
from jax.experimental.pallas import tpu_sc as plsc

_NC = 2
_NS = 16
_NWORKERS = _NC * _NS


def _sc_gather_rows(emb_table, idx, n_rows, hid):
    rows_per_w = n_rows // _NWORKERS

    mesh = plsc.VectorSubcoreMesh(core_axis_name="c", subcore_axis_name="s")

    @functools.partial(
        pl.kernel,
        mesh=mesh,
        out_type=jax.ShapeDtypeStruct((n_rows, hid), jnp.float32),
        scratch_types=[
            pltpu.VMEM((rows_per_w,), jnp.int32),
            pltpu.VMEM((rows_per_w, hid), jnp.float32),
            pltpu.SemaphoreType.DMA,
        ],
    )
    def gather_kernel(table_hbm, idx_hbm, out_hbm, idx_v, rows_v, sem):
        wid = lax.axis_index("s") * _NC + lax.axis_index("c")
        base = wid * rows_per_w
        pltpu.sync_copy(idx_hbm.at[pl.ds(base, rows_per_w)], idx_v)
        pltpu.async_copy(table_hbm.at[idx_v], rows_v, sem).wait()
        pltpu.sync_copy(rows_v, out_hbm.at[pl.ds(base, rows_per_w)])

    return gather_kernel(emb_table, idx)


def _tc_kernel(words_ref, zrow_ref, bbox_ref, bbox_t_ref, proj_ref, out_ref):
    nw = words_ref.shape[0]
    h = 128
    w = 128
    ngrp = nw // 16
    kdim = nw + 8

    rows_ext = jnp.concatenate([words_ref[...], zrow_ref[...]], axis=0)
    pt = lax.dot_general(
        proj_ref[...], rows_ext,
        (((0,), (1,)), ((), ())),
        preferred_element_type=jnp.float32,
    )

    bb = jnp.rint(bbox_ref[0])
    ws_c, we_c = bb[:, 0:1], bb[:, 2:3]
    bbt = jnp.rint(bbox_t_ref[0])
    hs_r, he_r = bbt[1:2, :], bbt[3:4, :]

    n_row = lax.broadcasted_iota(jnp.int32, (1, nw), 1)
    w_row = (jnp.int32(1) << (n_row & 15)).astype(jnp.float32)
    yy = lax.broadcasted_iota(jnp.int32, (h, nw), 0).astype(jnp.float32)
    y_mat = jnp.where((yy >= hs_r) & (yy < he_r), w_row, 0.0)

    n_col = lax.broadcasted_iota(jnp.int32, (nw, 1), 0)
    jlane = lax.broadcasted_iota(jnp.int32, (nw, ngrp * w), 1)
    xf = (jlane & (w - 1)).astype(jnp.float32)
    in_grp = (n_col >> 4) == (jlane >> 7)
    x_mat = jnp.where((xf >= ws_c) & (xf < we_c) & in_grp, 1.0, 0.0)

    s = jnp.dot(y_mat, x_mat, preferred_element_type=jnp.float32)
    e = (lax.bitcast_convert_type(s, jnp.int32) >> 23) - 127
    cand = e + (lax.broadcasted_iota(jnp.int32, (h, ngrp * w), 1) >> 7) * 16
    best = cand[:, 0:w]
    for g in range(1, ngrp):
        best = jnp.maximum(best, cand[:, g * w:(g + 1) * w])
    sel = jnp.where(best >= 0, best, nw)

    sel_flat = jnp.reshape(sel, (1, h * w))
    k_iota = lax.broadcasted_iota(jnp.int32, (kdim, h * w), 0)
    oh = (k_iota == sel_flat).astype(jnp.float32)
    out_t = jnp.dot(pt, oh, preferred_element_type=jnp.float32)
    out_ref[0] = jnp.reshape(out_t, (out_t.shape[0], h, w))


def kernel(img, input_ids, bbox, emb_table, proj_w, stride):
    b, _, img_h, img_w = img.shape
    h, w = img_h // 4, img_w // 4
    nw = input_ids.shape[1]
    hid = emb_table.shape[1]
    edim = proj_w.shape[1]

    n_flat = b * nw
    n_rows = -(-(n_flat + 8) // (8 * _NWORKERS)) * (8 * _NWORKERS)
    idx = jnp.concatenate([
        input_ids.reshape(-1).astype(jnp.int32),
        jnp.zeros((n_rows - n_flat,), jnp.int32),
    ])
    rows = _sc_gather_rows(emb_table, idx, n_rows, hid)

    bbox_s = (bbox / stride).astype(jnp.float32)
    bbox_t = jnp.transpose(bbox_s, (0, 2, 1))

    out_flat = pl.pallas_call(
        _tc_kernel,
        grid=(b,),
        in_specs=[
            pl.BlockSpec((nw, hid), lambda i: (i, 0)),
            pl.BlockSpec((8, hid), lambda i: (n_flat // 8, 0)),
            pl.BlockSpec((1, nw, 4), lambda i: (i, 0, 0)),
            pl.BlockSpec((1, 4, nw), lambda i: (i, 0, 0)),
            pl.BlockSpec((hid, edim), lambda i: (0, 0)),
        ],
        out_specs=pl.BlockSpec((1, edim, h, w), lambda i: (i, 0, 0, 0)),
        out_shape=jax.ShapeDtypeStruct((b, edim, h, w), jnp.float32),
    )(rows, rows, bbox_s, bbox_t, proj_w)

    return out_flat


# speedup vs baseline: 97.5049x; 1.1099x over previous
import functools

import jax
import jax.numpy as jnp
from jax import lax
from jax.experimental import pallas as pl
from jax.experimental.pallas import tpu as pltpu
from jax.experimental.pallas import tpu_sc as plsc

_NC = 1
_NS = 16
_NWORKERS = _NC * _NS


def _sc_gather_rows(emb_table, idx, n_rows, hid):
    rows_per_w = n_rows // _NWORKERS

    mesh = plsc.VectorSubcoreMesh(
        core_axis_name="c", subcore_axis_name="s", num_cores=_NC)

    @functools.partial(
        pl.kernel,
        mesh=mesh,
        out_type=jax.ShapeDtypeStruct((n_rows, hid), jnp.float32),
        scratch_types=[
            pltpu.VMEM((rows_per_w,), jnp.int32),
            pltpu.VMEM((rows_per_w, hid), jnp.float32),
            pltpu.SemaphoreType.DMA,
        ],
    )
    def gather_kernel(table_hbm, idx_hbm, out_hbm, idx_v, rows_v, sem):
        wid = lax.axis_index("s") * _NC + lax.axis_index("c")
        base = wid * rows_per_w
        pltpu.sync_copy(idx_hbm.at[pl.ds(base, rows_per_w)], idx_v)
        pltpu.async_copy(table_hbm.at[idx_v], rows_v, sem).wait()
        pltpu.sync_copy(rows_v, out_hbm.at[pl.ds(base, rows_per_w)])

    return gather_kernel(emb_table, idx)


def _tc_kernel(words_ref, zrow_ref, bbox_ref, bbox_t_ref, proj_ref, out_ref):
    nw = words_ref.shape[0]
    h = 128
    w = 128
    ngrp = nw // 16
    kdim = nw + 8

    rows_ext = jnp.concatenate([words_ref[...], zrow_ref[...]], axis=0)
    pt = lax.dot_general(
        proj_ref[...], rows_ext,
        (((0,), (1,)), ((), ())),
        preferred_element_type=jnp.float32,
    )

    bb = jnp.rint(bbox_ref[0])
    ws_c, we_c = bb[:, 0:1], bb[:, 2:3]
    bbt = jnp.rint(bbox_t_ref[0])
    hs_r, he_r = bbt[1:2, :], bbt[3:4, :]

    n_row = lax.broadcasted_iota(jnp.int32, (1, nw), 1)
    w_row = (jnp.int32(1) << (n_row & 15)).astype(jnp.float32)
    yy = lax.broadcasted_iota(jnp.int32, (h, nw), 0).astype(jnp.float32)
    y_mat = jnp.where((yy >= hs_r) & (yy < he_r), w_row, 0.0)

    n_col = lax.broadcasted_iota(jnp.int32, (nw, 1), 0)
    jlane = lax.broadcasted_iota(jnp.int32, (nw, ngrp * w), 1)
    xf = (jlane & (w - 1)).astype(jnp.float32)
    in_grp = (n_col >> 4) == (jlane >> 7)
    x_mat = jnp.where((xf >= ws_c) & (xf < we_c) & in_grp, 1.0, 0.0)

    s = jnp.dot(y_mat, x_mat, preferred_element_type=jnp.float32)
    e = (lax.bitcast_convert_type(s, jnp.int32) >> 23) - 127
    cand = e + (lax.broadcasted_iota(jnp.int32, (h, ngrp * w), 1) >> 7) * 16
    best = cand[:, 0:w]
    for g in range(1, ngrp):
        best = jnp.maximum(best, cand[:, g * w:(g + 1) * w])
    sel = jnp.where(best >= 0, best, nw)

    sel_flat = jnp.reshape(sel, (1, h * w))
    k_iota = lax.broadcasted_iota(jnp.int32, (kdim, h * w), 0)
    oh = (k_iota == sel_flat).astype(jnp.float32)
    out_t = jnp.dot(pt, oh, preferred_element_type=jnp.float32)
    out_ref[0] = jnp.reshape(out_t, (out_t.shape[0], h, w))


def kernel(img, input_ids, bbox, emb_table, proj_w, stride):
    b, _, img_h, img_w = img.shape
    h, w = img_h // 4, img_w // 4
    nw = input_ids.shape[1]
    hid = emb_table.shape[1]
    edim = proj_w.shape[1]

    n_flat = b * nw
    n_rows = -(-(n_flat + 8) // (8 * _NWORKERS)) * (8 * _NWORKERS)
    idx = jnp.concatenate([
        input_ids.reshape(-1).astype(jnp.int32),
        jnp.zeros((n_rows - n_flat,), jnp.int32),
    ])
    rows = _sc_gather_rows(emb_table, idx, n_rows, hid)

    bbox_s = (bbox / stride).astype(jnp.float32)
    bbox_t = jnp.transpose(bbox_s, (0, 2, 1))

    out_flat = pl.pallas_call(
        _tc_kernel,
        grid=(b,),
        in_specs=[
            pl.BlockSpec((nw, hid), lambda i: (i, 0)),
            pl.BlockSpec((8, hid), lambda i: (n_flat // 8, 0)),
            pl.BlockSpec((1, nw, 4), lambda i: (i, 0, 0)),
            pl.BlockSpec((1, 4, nw), lambda i: (i, 0, 0)),
            pl.BlockSpec((hid, edim), lambda i: (0, 0)),
        ],
        out_specs=pl.BlockSpec((1, edim, h, w), lambda i: (i, 0, 0, 0)),
        out_shape=jax.ShapeDtypeStruct((b, edim, h, w), jnp.float32),
    )(rows, rows, bbox_s, bbox_t, proj_w)

    return out_flat


# speedup vs baseline: 114.8358x; 1.1777x over previous
import functools

import jax
import jax.numpy as jnp
from jax import lax
from jax.experimental import pallas as pl
from jax.experimental.pallas import tpu as pltpu
from jax.experimental.pallas import tpu_sc as plsc

_NS = 16
_PAD = 16


def _sc_gather_rows(emb_table, ids_flat, hid):
    n_flat = ids_flat.shape[0]
    rows_per_w = n_flat // _NS

    mesh = plsc.VectorSubcoreMesh(
        core_axis_name="c", subcore_axis_name="s", num_cores=1)

    @functools.partial(
        pl.kernel,
        mesh=mesh,
        out_type=jax.ShapeDtypeStruct((n_flat + _PAD, hid), jnp.float32),
        scratch_types=[
            pltpu.VMEM((rows_per_w,), jnp.int32),
            pltpu.VMEM((rows_per_w, hid), jnp.float32),
            pltpu.VMEM((_PAD,), jnp.int32),
            pltpu.VMEM((_PAD, hid), jnp.float32),
            pltpu.SemaphoreType.DMA,
        ],
    )
    def gather_kernel(table_hbm, idx_hbm, out_hbm, idx_v, rows_v, idx0_v,
                      rows0_v, sem):
        wid = lax.axis_index("s")
        base = wid * rows_per_w
        pltpu.sync_copy(idx_hbm.at[pl.ds(base, rows_per_w)], idx_v)
        pltpu.async_copy(table_hbm.at[idx_v], rows_v, sem).wait()
        pltpu.sync_copy(rows_v, out_hbm.at[pl.ds(base, rows_per_w)])

        @pl.when(wid == 0)
        def _():
            idx0_v[...] = jnp.zeros((_PAD,), jnp.int32)
            pltpu.async_copy(table_hbm.at[idx0_v], rows0_v, sem).wait()
            pltpu.sync_copy(rows0_v, out_hbm.at[pl.ds(n_flat, _PAD)])

    return gather_kernel(emb_table, ids_flat)


def _tc_kernel(words_ref, zrow_ref, bbox_ref, proj_ref, out_ref):
    nw = words_ref.shape[0]
    h, w = out_ref.shape[2], out_ref.shape[3]
    ngrp = nw // 16
    kdim = nw + 8

    rows_ext = jnp.concatenate([words_ref[...], zrow_ref[...]], axis=0)
    pt = lax.dot_general(
        proj_ref[...], rows_ext,
        (((0,), (1,)), ((), ())),
        preferred_element_type=jnp.float32,
    )

    bb = jnp.rint(bbox_ref[0])
    ws_c, we_c = bb[:, 0:1], bb[:, 2:3]
    hs_c, he_c = bb[:, 1:2], bb[:, 3:4]

    n_col = lax.broadcasted_iota(jnp.int32, (nw, 1), 0)
    w_col = (jnp.int32(1) << (n_col & 15)).astype(jnp.float32)
    yy = lax.broadcasted_iota(jnp.int32, (nw, h), 1).astype(jnp.float32)
    yt_mat = jnp.where((yy >= hs_c) & (yy < he_c), w_col, 0.0)

    jlane = lax.broadcasted_iota(jnp.int32, (nw, ngrp * w), 1)
    xf = (jlane & (w - 1)).astype(jnp.float32)
    in_grp = (n_col >> 4) == (jlane >> 7)
    x_mat = jnp.where((xf >= ws_c) & (xf < we_c) & in_grp, 1.0, 0.0)

    s = lax.dot_general(
        yt_mat, x_mat,
        (((0,), (0,)), ((), ())),
        preferred_element_type=jnp.float32,
    )
    e = (lax.bitcast_convert_type(s, jnp.int32) >> 23) - 127
    cand = e + (lax.broadcasted_iota(jnp.int32, (h, ngrp * w), 1) >> 7) * 16
    best = cand[:, 0:w]
    for g in range(1, ngrp):
        best = jnp.maximum(best, cand[:, g * w:(g + 1) * w])
    sel = jnp.where(best >= 0, best, nw)

    sel_flat = jnp.reshape(sel, (1, h * w))
    k_iota = lax.broadcasted_iota(jnp.int32, (kdim, h * w), 0)
    oh = (k_iota == sel_flat).astype(jnp.float32)
    out_t = jnp.dot(pt, oh, preferred_element_type=jnp.float32)
    out_ref[0] = jnp.reshape(out_t, (out_t.shape[0], h, w))


def kernel(img, input_ids, bbox, emb_table, proj_w, stride):
    b, _, img_h, img_w = img.shape
    h, w = img_h // 4, img_w // 4
    nw = input_ids.shape[1]
    hid = emb_table.shape[1]
    edim = proj_w.shape[1]
    n_flat = b * nw

    ids_flat = input_ids.reshape(-1).astype(jnp.int32)
    rows = _sc_gather_rows(emb_table, ids_flat, hid)

    bbox_s = (bbox / stride).astype(jnp.float32)

    return pl.pallas_call(
        _tc_kernel,
        grid=(b,),
        in_specs=[
            pl.BlockSpec((nw, hid), lambda i: (i, 0)),
            pl.BlockSpec((8, hid), lambda i: (n_flat // 8, 0)),
            pl.BlockSpec((1, nw, 4), lambda i: (i, 0, 0)),
            pl.BlockSpec((hid, edim), lambda i: (0, 0)),
        ],
        out_specs=pl.BlockSpec((1, edim, h, w), lambda i: (i, 0, 0, 0)),
        out_shape=jax.ShapeDtypeStruct((b, edim, h, w), jnp.float32),
    )(rows, rows, bbox_s, proj_w)
